# baseline (speedup 1.0000x reference)
import functools

import jax
import jax.numpy as jnp
from jax.experimental import pallas as pl

_NEG = -1e30


def _cnt_kernel(cha_ref, spa_ref, c1_ref, c2_ref):
    src1 = cha_ref[0:1, :]
    dst1 = cha_ref[1:2, :]
    i64 = jax.lax.broadcasted_iota(jnp.int32, (64, 2048), 0)
    a1 = (dst1 == i64).astype(jnp.float32)
    b1 = (src1 == i64).astype(jnp.float32)
    c1_ref[...] = jax.lax.dot_general(
        a1, b1, (((1,), (1,)), ((), ())), preferred_element_type=jnp.float32)
    src2 = spa_ref[0:1, :]
    dst2 = spa_ref[1:2, :]
    i36 = jax.lax.broadcasted_iota(jnp.int32, (36, 1024), 0)
    a2 = (dst2 == i36).astype(jnp.float32)
    b2 = (src2 == i36).astype(jnp.float32)
    c2_ref[...] = jax.lax.dot_general(
        a2, b2, (((1,), (1,)), ((), ())), preferred_element_type=jnp.float32)


def _gat_dense(x, W, Al, Ar, SD, cnt, H, D):
    nb, n, f = x.shape
    h = jax.lax.dot_general(
        x, W, (((2,), (1,)), ((), ())), preferred_element_type=jnp.float32)
    el = jax.lax.dot_general(
        h, Al, (((2,), (0,)), ((), ())), preferred_element_type=jnp.float32)
    er = jax.lax.dot_general(
        h, Ar, (((2,), (0,)), ((), ())), preferred_element_type=jnp.float32)
    elt = el.transpose(0, 2, 1)
    ert = er.transpose(0, 2, 1)
    e = elt[:, :, None, :] + ert[:, :, :, None]
    e = jnp.maximum(e, 0.2 * e)
    ex = jnp.exp(e) * cnt[None, None, :, :]
    denom = jnp.sum(ex, axis=-1)
    rden = jnp.where(denom > 0, 1.0 / denom, 0.0)
    rdf = jax.lax.dot_general(
        rden.transpose(0, 2, 1), SD, (((2,), (0,)), ((), ())),
        preferred_element_type=jnp.float32)
    parts = []
    for hd in range(H):
        parts.append(jax.lax.dot_general(
            ex[:, hd], h[:, :, hd * D:(hd + 1) * D],
            (((2,), (1,)), ((0,), (0,))),
            preferred_element_type=jnp.float32))
    return jnp.concatenate(parts, axis=-1) * rdf


def _main_kernel(nd_ref, c1_ref, c2_ref, wc_ref, alc_ref, arc_ref, bc_ref,
                 sdc_ref, ws_ref, als_ref, ars_ref, bs_ref, sds_ref, out_ref):
    x = nd_ref[...]
    r1 = _gat_dense(x, wc_ref[...], alc_ref[...], arc_ref[...],
                    sdc_ref[...], c1_ref[...], 6, 6)
    r1 = r1 + bc_ref[...]
    y = r1.transpose(0, 2, 1)
    r2 = _gat_dense(y, ws_ref[...], als_ref[...], ars_ref[...],
                    sds_ref[...], c2_ref[...], 8, 8)
    r2 = r2 + bs_ref[...]
    out_ref[...] = r2.transpose(0, 2, 1)


def _whole(shape):
    nd = len(shape)
    return pl.BlockSpec(shape, lambda i: (0,) * nd)


@functools.partial(jax.jit, static_argnames=("interpret",))
def _run(ndata, cha_con, spa_con, W_cha, al_cha, ar_cha, b_cha,
         W_spa, al_spa, ar_spa, b_spa, interpret=False):
    nd = ndata.reshape(512, 64, 36)
    cnt1, cnt2 = pl.pallas_call(
        _cnt_kernel,
        out_shape=[jax.ShapeDtypeStruct((64, 64), jnp.float32),
                   jax.ShapeDtypeStruct((36, 36), jnp.float32)],
        interpret=interpret,
    )(cha_con.reshape(2, 2048), spa_con.reshape(2, 1024))

    eye6 = jnp.eye(6, dtype=jnp.float32)
    alc = (al_cha[0][:, :, None] * eye6[:, None, :]).reshape(36, 6)
    arc = (ar_cha[0][:, :, None] * eye6[:, None, :]).reshape(36, 6)
    eye8 = jnp.eye(8, dtype=jnp.float32)
    als = (al_spa[0][:, :, None] * eye8[:, None, :]).reshape(64, 8)
    ars = (ar_spa[0][:, :, None] * eye8[:, None, :]).reshape(64, 8)
    sdc = jnp.repeat(eye6, 6, axis=1)
    sds = jnp.repeat(eye8, 8, axis=1)

    NB = 32
    out = pl.pallas_call(
        _main_kernel,
        grid=(512 // NB,),
        in_specs=[
            pl.BlockSpec((NB, 64, 36), lambda i: (i, 0, 0)),
            _whole((64, 64)), _whole((36, 36)),
            _whole((36, 36)), _whole((36, 6)), _whole((36, 6)), _whole((1, 36)),
            _whole((6, 36)),
            _whole((64, 64)), _whole((64, 8)), _whole((64, 8)), _whole((1, 64)),
            _whole((8, 64)),
        ],
        out_specs=pl.BlockSpec((NB, 64, 36), lambda i: (i, 0, 0)),
        out_shape=jax.ShapeDtypeStruct((512, 64, 36), jnp.float32),
        interpret=interpret,
    )(nd, cnt1, cnt2, W_cha, alc, arc, b_cha.reshape(1, 36), sdc,
      W_spa, als, ars, b_spa.reshape(1, 64), sds)
    return out.reshape(4, 8, 16, 64, 36)


def kernel(ndata, cha_con, spa_con, W_cha, al_cha, ar_cha, b_cha,
           W_spa, al_spa, ar_spa, b_spa):
    return _run(ndata, cha_con, spa_con, W_cha, al_cha, ar_cha, b_cha,
                W_spa, al_spa, ar_spa, b_spa)


# speedup vs baseline: 1.0513x; 1.0513x over previous
import functools

import jax
import jax.numpy as jnp
from jax.experimental import pallas as pl

_NEG = -1e30


def _cnt_kernel(cha_ref, spa_ref, c1_ref, c2_ref):
    src1 = cha_ref[0:1, :]
    dst1 = cha_ref[1:2, :]
    i64 = jax.lax.broadcasted_iota(jnp.int32, (64, 2048), 0)
    a1 = (dst1 == i64).astype(jnp.float32)
    b1 = (src1 == i64).astype(jnp.float32)
    c1_ref[...] = jax.lax.dot_general(
        a1, b1, (((1,), (1,)), ((), ())), preferred_element_type=jnp.float32)
    src2 = spa_ref[0:1, :]
    dst2 = spa_ref[1:2, :]
    i36 = jax.lax.broadcasted_iota(jnp.int32, (36, 1024), 0)
    a2 = (dst2 == i36).astype(jnp.float32)
    b2 = (src2 == i36).astype(jnp.float32)
    c2_ref[...] = jax.lax.dot_general(
        a2, b2, (((1,), (1,)), ((), ())), preferred_element_type=jnp.float32)


def _gat_dense(x, W, Al, Ar, SD, cnt, H, D):
    nb, n, f = x.shape
    h = jax.lax.dot_general(
        x, W, (((2,), (1,)), ((), ())), preferred_element_type=jnp.float32)
    el = jax.lax.dot_general(
        h, Al, (((2,), (0,)), ((), ())), preferred_element_type=jnp.float32)
    er = jax.lax.dot_general(
        h, Ar, (((2,), (0,)), ((), ())), preferred_element_type=jnp.float32)
    elt = el.transpose(0, 2, 1)
    ert = er.transpose(0, 2, 1)
    e = elt[:, :, None, :] + ert[:, :, :, None]
    e = jnp.maximum(e, 0.2 * e)
    ex = jnp.exp(e) * cnt[None, None, :, :]
    denom = jnp.sum(ex, axis=-1)
    rden = jnp.where(denom > 0, 1.0 / denom, 0.0)
    parts = []
    for hd in range(H):
        parts.append(rden[:, hd, :, None] * jax.lax.dot_general(
            ex[:, hd], h[:, :, hd * D:(hd + 1) * D],
            (((2,), (1,)), ((0,), (0,))),
            preferred_element_type=jnp.float32))
    return jnp.concatenate(parts, axis=-1)


def _main_kernel(nd_ref, c1_ref, c2_ref, wc_ref, alc_ref, arc_ref, bc_ref,
                 sdc_ref, ws_ref, als_ref, ars_ref, bs_ref, sds_ref, out_ref):
    x = nd_ref[...]
    r1 = _gat_dense(x, wc_ref[...], alc_ref[...], arc_ref[...],
                    sdc_ref[...], c1_ref[...], 6, 6)
    r1 = r1 + bc_ref[...]
    y = r1.transpose(0, 2, 1)
    r2 = _gat_dense(y, ws_ref[...], als_ref[...], ars_ref[...],
                    sds_ref[...], c2_ref[...], 8, 8)
    r2 = r2 + bs_ref[...]
    out_ref[...] = r2.transpose(0, 2, 1)


def _whole(shape):
    nd = len(shape)
    return pl.BlockSpec(shape, lambda i: (0,) * nd)


@functools.partial(jax.jit, static_argnames=("interpret",))
def _run(ndata, cha_con, spa_con, W_cha, al_cha, ar_cha, b_cha,
         W_spa, al_spa, ar_spa, b_spa, interpret=False):
    nd = ndata.reshape(512, 64, 36)
    cnt1, cnt2 = pl.pallas_call(
        _cnt_kernel,
        out_shape=[jax.ShapeDtypeStruct((64, 64), jnp.float32),
                   jax.ShapeDtypeStruct((36, 36), jnp.float32)],
        interpret=interpret,
    )(cha_con.reshape(2, 2048), spa_con.reshape(2, 1024))

    eye6 = jnp.eye(6, dtype=jnp.float32)
    alc = (al_cha[0][:, :, None] * eye6[:, None, :]).reshape(36, 6)
    arc = (ar_cha[0][:, :, None] * eye6[:, None, :]).reshape(36, 6)
    eye8 = jnp.eye(8, dtype=jnp.float32)
    als = (al_spa[0][:, :, None] * eye8[:, None, :]).reshape(64, 8)
    ars = (ar_spa[0][:, :, None] * eye8[:, None, :]).reshape(64, 8)
    sdc = jnp.repeat(eye6, 6, axis=1)
    sds = jnp.repeat(eye8, 8, axis=1)

    NB = 32
    out = pl.pallas_call(
        _main_kernel,
        grid=(512 // NB,),
        in_specs=[
            pl.BlockSpec((NB, 64, 36), lambda i: (i, 0, 0)),
            _whole((64, 64)), _whole((36, 36)),
            _whole((36, 36)), _whole((36, 6)), _whole((36, 6)), _whole((1, 36)),
            _whole((6, 36)),
            _whole((64, 64)), _whole((64, 8)), _whole((64, 8)), _whole((1, 64)),
            _whole((8, 64)),
        ],
        out_specs=pl.BlockSpec((NB, 64, 36), lambda i: (i, 0, 0)),
        out_shape=jax.ShapeDtypeStruct((512, 64, 36), jnp.float32),
        interpret=interpret,
    )(nd, cnt1, cnt2, W_cha, alc, arc, b_cha.reshape(1, 36), sdc,
      W_spa, als, ars, b_spa.reshape(1, 64), sds)
    return out.reshape(4, 8, 16, 64, 36)


def kernel(ndata, cha_con, spa_con, W_cha, al_cha, ar_cha, b_cha,
           W_spa, al_spa, ar_spa, b_spa):
    return _run(ndata, cha_con, spa_con, W_cha, al_cha, ar_cha, b_cha,
                W_spa, al_spa, ar_spa, b_spa)


# speedup vs baseline: 1.0741x; 1.0217x over previous
import functools

import jax
import jax.numpy as jnp
from jax.experimental import pallas as pl

_NEG = -1e30


def _cnt_kernel(cha_ref, spa_ref, c1_ref, c2_ref):
    src1 = cha_ref[0:1, :]
    dst1 = cha_ref[1:2, :]
    i64 = jax.lax.broadcasted_iota(jnp.int32, (64, 2048), 0)
    a1 = (dst1 == i64).astype(jnp.float32)
    b1 = (src1 == i64).astype(jnp.float32)
    c1_ref[...] = jax.lax.dot_general(
        a1, b1, (((1,), (1,)), ((), ())), preferred_element_type=jnp.float32)
    src2 = spa_ref[0:1, :]
    dst2 = spa_ref[1:2, :]
    i36 = jax.lax.broadcasted_iota(jnp.int32, (36, 1024), 0)
    a2 = (dst2 == i36).astype(jnp.float32)
    b2 = (src2 == i36).astype(jnp.float32)
    c2_ref[...] = jax.lax.dot_general(
        a2, b2, (((1,), (1,)), ((), ())), preferred_element_type=jnp.float32)


def _gat_dense(x, W, Al, Ar, SD, cnt, H, D):
    nb, n, f = x.shape
    h = jax.lax.dot_general(
        x, W, (((2,), (1,)), ((), ())), preferred_element_type=jnp.float32)
    el = jax.lax.dot_general(
        h, Al, (((2,), (0,)), ((), ())), preferred_element_type=jnp.float32)
    er = jax.lax.dot_general(
        h, Ar, (((2,), (0,)), ((), ())), preferred_element_type=jnp.float32)
    elt = el.transpose(0, 2, 1)
    ert = er.transpose(0, 2, 1)
    e = elt[:, :, None, :] + ert[:, :, :, None]
    e = jnp.maximum(e, 0.2 * e)
    ex = jnp.exp(e) * cnt[None, None, :, :]
    denom = jnp.sum(ex, axis=-1)
    rden = jnp.where(denom > 0, 1.0 / denom, 0.0)
    parts = []
    for hd in range(H):
        parts.append(rden[:, hd, :, None] * jax.lax.dot_general(
            ex[:, hd], h[:, :, hd * D:(hd + 1) * D],
            (((2,), (1,)), ((0,), (0,))),
            preferred_element_type=jnp.float32))
    return jnp.concatenate(parts, axis=-1)


def _main_kernel(nd_ref, c1_ref, c2_ref, wc_ref, alc_ref, arc_ref, bc_ref,
                 sdc_ref, ws_ref, als_ref, ars_ref, bs_ref, sds_ref, out_ref):
    x = nd_ref[...]
    r1 = _gat_dense(x, wc_ref[...], alc_ref[...], arc_ref[...],
                    sdc_ref[...], c1_ref[...], 6, 6)
    r1 = r1 + bc_ref[...]
    y = r1.transpose(0, 2, 1)
    r2 = _gat_dense(y, ws_ref[...], als_ref[...], ars_ref[...],
                    sds_ref[...], c2_ref[...], 8, 8)
    r2 = r2 + bs_ref[...]
    out_ref[...] = r2.transpose(0, 2, 1)


def _whole(shape):
    nd = len(shape)
    return pl.BlockSpec(shape, lambda i: (0,) * nd)


@functools.partial(jax.jit, static_argnames=("interpret",))
def _run(ndata, cha_con, spa_con, W_cha, al_cha, ar_cha, b_cha,
         W_spa, al_spa, ar_spa, b_spa, interpret=False):
    nd = ndata.reshape(512, 64, 36)
    cnt1, cnt2 = pl.pallas_call(
        _cnt_kernel,
        out_shape=[jax.ShapeDtypeStruct((64, 64), jnp.float32),
                   jax.ShapeDtypeStruct((36, 36), jnp.float32)],
        interpret=interpret,
    )(cha_con.reshape(2, 2048), spa_con.reshape(2, 1024))

    eye6 = jnp.eye(6, dtype=jnp.float32)
    alc = (al_cha[0][:, :, None] * eye6[:, None, :]).reshape(36, 6)
    arc = (ar_cha[0][:, :, None] * eye6[:, None, :]).reshape(36, 6)
    eye8 = jnp.eye(8, dtype=jnp.float32)
    als = (al_spa[0][:, :, None] * eye8[:, None, :]).reshape(64, 8)
    ars = (ar_spa[0][:, :, None] * eye8[:, None, :]).reshape(64, 8)
    sdc = jnp.repeat(eye6, 6, axis=1)
    sds = jnp.repeat(eye8, 8, axis=1)

    NB = 64
    out = pl.pallas_call(
        _main_kernel,
        grid=(512 // NB,),
        in_specs=[
            pl.BlockSpec((NB, 64, 36), lambda i: (i, 0, 0)),
            _whole((64, 64)), _whole((36, 36)),
            _whole((36, 36)), _whole((36, 6)), _whole((36, 6)), _whole((1, 36)),
            _whole((6, 36)),
            _whole((64, 64)), _whole((64, 8)), _whole((64, 8)), _whole((1, 64)),
            _whole((8, 64)),
        ],
        out_specs=pl.BlockSpec((NB, 64, 36), lambda i: (i, 0, 0)),
        out_shape=jax.ShapeDtypeStruct((512, 64, 36), jnp.float32),
        interpret=interpret,
    )(nd, cnt1, cnt2, W_cha, alc, arc, b_cha.reshape(1, 36), sdc,
      W_spa, als, ars, b_spa.reshape(1, 64), sds)
    return out.reshape(4, 8, 16, 64, 36)


def kernel(ndata, cha_con, spa_con, W_cha, al_cha, ar_cha, b_cha,
           W_spa, al_spa, ar_spa, b_spa):
    return _run(ndata, cha_con, spa_con, W_cha, al_cha, ar_cha, b_cha,
                W_spa, al_spa, ar_spa, b_spa)


# speedup vs baseline: 1.0763x; 1.0020x over previous
import functools

import jax
import jax.numpy as jnp
from jax.experimental import pallas as pl

_NEG = -1e30


def _edge_cnt(con, n):
    e = con.shape[1]
    iota = jax.lax.broadcasted_iota(jnp.int32, (n, e), 0)
    a = (con[1:2, :] == iota).astype(jnp.float32)
    b = (con[0:1, :] == iota).astype(jnp.float32)
    return jax.lax.dot_general(
        a, b, (((1,), (1,)), ((), ())), preferred_element_type=jnp.float32)


def _gat_dense(x, W, Al, Ar, SD, cnt, H, D):
    nb, n, f = x.shape
    h = jax.lax.dot_general(
        x, W, (((2,), (1,)), ((), ())), preferred_element_type=jnp.float32)
    el = jax.lax.dot_general(
        h, Al, (((2,), (0,)), ((), ())), preferred_element_type=jnp.float32)
    er = jax.lax.dot_general(
        h, Ar, (((2,), (0,)), ((), ())), preferred_element_type=jnp.float32)
    elt = el.transpose(0, 2, 1)
    ert = er.transpose(0, 2, 1)
    e = elt[:, :, None, :] + ert[:, :, :, None]
    e = jnp.maximum(e, 0.2 * e)
    ex = jnp.exp(e) * cnt[None, None, :, :]
    denom = jnp.sum(ex, axis=-1)
    rden = jnp.where(denom > 0, 1.0 / denom, 0.0)
    parts = []
    for hd in range(H):
        parts.append(rden[:, hd, :, None] * jax.lax.dot_general(
            ex[:, hd], h[:, :, hd * D:(hd + 1) * D],
            (((2,), (1,)), ((0,), (0,))),
            preferred_element_type=jnp.float32))
    return jnp.concatenate(parts, axis=-1)


def _main_kernel(nd_ref, cha_ref, spa_ref, wc_ref, alc_ref, arc_ref, bc_ref,
                 sdc_ref, ws_ref, als_ref, ars_ref, bs_ref, sds_ref, out_ref):
    cnt1 = _edge_cnt(cha_ref[...], 64)
    cnt2 = _edge_cnt(spa_ref[...], 36)
    x = nd_ref[...]
    r1 = _gat_dense(x, wc_ref[...], alc_ref[...], arc_ref[...],
                    sdc_ref[...], cnt1, 6, 6)
    r1 = r1 + bc_ref[...]
    y = r1.transpose(0, 2, 1)
    r2 = _gat_dense(y, ws_ref[...], als_ref[...], ars_ref[...],
                    sds_ref[...], cnt2, 8, 8)
    r2 = r2 + bs_ref[...]
    out_ref[...] = r2.transpose(0, 2, 1)


def _whole(shape):
    nd = len(shape)
    return pl.BlockSpec(shape, lambda i: (0,) * nd)


@functools.partial(jax.jit, static_argnames=("interpret",))
def _run(ndata, cha_con, spa_con, W_cha, al_cha, ar_cha, b_cha,
         W_spa, al_spa, ar_spa, b_spa, interpret=False):
    nd = ndata.reshape(512, 64, 36)
    eye6 = jnp.eye(6, dtype=jnp.float32)
    alc = (al_cha[0][:, :, None] * eye6[:, None, :]).reshape(36, 6)
    arc = (ar_cha[0][:, :, None] * eye6[:, None, :]).reshape(36, 6)
    eye8 = jnp.eye(8, dtype=jnp.float32)
    als = (al_spa[0][:, :, None] * eye8[:, None, :]).reshape(64, 8)
    ars = (ar_spa[0][:, :, None] * eye8[:, None, :]).reshape(64, 8)
    sdc = jnp.repeat(eye6, 6, axis=1)
    sds = jnp.repeat(eye8, 8, axis=1)

    NB = 64
    out = pl.pallas_call(
        _main_kernel,
        grid=(512 // NB,),
        in_specs=[
            pl.BlockSpec((NB, 64, 36), lambda i: (i, 0, 0)),
            _whole((2, 2048)), _whole((2, 1024)),
            _whole((36, 36)), _whole((36, 6)), _whole((36, 6)), _whole((1, 36)),
            _whole((6, 36)),
            _whole((64, 64)), _whole((64, 8)), _whole((64, 8)), _whole((1, 64)),
            _whole((8, 64)),
        ],
        out_specs=pl.BlockSpec((NB, 64, 36), lambda i: (i, 0, 0)),
        out_shape=jax.ShapeDtypeStruct((512, 64, 36), jnp.float32),
        interpret=interpret,
    )(nd, cha_con.reshape(2, 2048), spa_con.reshape(2, 1024),
      W_cha, alc, arc, b_cha.reshape(1, 36), sdc,
      W_spa, als, ars, b_spa.reshape(1, 64), sds)
    return out.reshape(4, 8, 16, 64, 36)


def kernel(ndata, cha_con, spa_con, W_cha, al_cha, ar_cha, b_cha,
           W_spa, al_spa, ar_spa, b_spa):
    return _run(ndata, cha_con, spa_con, W_cha, al_cha, ar_cha, b_cha,
                W_spa, al_spa, ar_spa, b_spa)
